# baseline (speedup 1.0000x reference)
import dataclasses
import functools

import jax
import jax.numpy as jnp
from jax import lax
from jax.experimental import pallas as pl
from jax.experimental.pallas import tpu as pltpu
from jax.experimental.pallas import tpu_sc as plsc

_A = 4
_K = 6
_B = 1024
_L = 4096
_NBINS = _A ** _K
_OUT_LEN = _L - _K + 1

_R1 = 16

_NW = 32
_ROWS_PER_WORKER = _B // _NW

_LANES = 16
_TAIL_START = (_OUT_LEN // _LANES) * _LANES
_TAIL_N = _OUT_LEN - _TAIL_START

_RB = 4
_NBATCH = _ROWS_PER_WORKER // _RB


def _shup(x, d, axis):
    return pltpu.roll(x, x.shape[axis] - d, axis)


def _kmer_tc_body(x_ref, o_ref):
    x0 = x_ref[:, 0, :]
    x1 = x_ref[:, 1, :]
    x2 = x_ref[:, 2, :]
    x3 = x_ref[:, 3, :]
    m01 = jnp.maximum(x0, x1)
    b01 = m01 > x0
    m23 = jnp.maximum(x2, x3)
    h = m23 > m01
    i01 = jnp.where(b01, 1, 0)
    i23 = jnp.where(m23 > x2, 3, 2)
    c = jnp.where(h, i23, i01)
    y1 = c * 4 + _shup(c, 1, 1)
    y2 = y1 * 16 + _shup(y1, 2, 1)
    km = y2 * 16 + _shup(y1, 4, 1)
    o_ref[...] = km


_NCHUNK = 4
_CB = _B // _NCHUNK


def _kmer_tc(xt, c0, nrows):
    return pl.pallas_call(
        _kmer_tc_body,
        grid=(nrows // _R1,),
        in_specs=[pl.BlockSpec((_R1, _A, _L),
                               lambda i, c=c0: (i + c // _R1, 0, 0))],
        out_specs=pl.BlockSpec((_R1, _L), lambda i: (i, 0)),
        out_shape=jax.ShapeDtypeStruct((nrows, _L), jnp.int32),
        compiler_params=pltpu.CompilerParams(
            dimension_semantics=("parallel",)),
    )(xt)


def _hist_sc(kmers):
    mesh = plsc.VectorSubcoreMesh(core_axis_name="c", subcore_axis_name="s")
    cp = pltpu.CompilerParams()
    if "needs_layout_passes" in pltpu.CompilerParams.__dataclass_fields__:
        cp = dataclasses.replace(cp, needs_layout_passes=False)

    @functools.partial(
        pl.kernel,
        compiler_params=cp,
        out_type=jax.ShapeDtypeStruct((_B, _NBINS), jnp.float32),
        mesh=mesh,
        scratch_types=[
            pltpu.VMEM((_RB, _L), jnp.int32),
            pltpu.VMEM((_RB, _L), jnp.int32),
            pltpu.VMEM((_RB, _NBINS), jnp.float32),
            pltpu.VMEM((_RB, _NBINS), jnp.float32),
            pltpu.SemaphoreType.DMA,
            pltpu.SemaphoreType.DMA,
            pltpu.SemaphoreType.DMA,
            pltpu.SemaphoreType.DMA,
        ],
    )
    def k(kmers_hbm, out_hbm, kb0, kb1, h0, h1, si0, si1, so0, so1):
        wid = lax.axis_index("s") * 2 + lax.axis_index("c")
        base = wid * _ROWS_PER_WORKER
        kbs = (kb0, kb1)
        hs = (h0, h1)
        sis = (si0, si1)
        sos = (so0, so1)
        zeros16 = jnp.zeros((_LANES,), jnp.float32)
        tail_valid = lax.iota(jnp.int32, _LANES) < _TAIL_N

        pltpu.async_copy(kmers_hbm.at[pl.ds(base, _RB)], kb0, si0)
        pltpu.async_copy(kmers_hbm.at[pl.ds(base + _RB, _RB)], kb1, si1)

        @pl.loop(0, _NBATCH, step=2)
        def _batch(bb):
            for p in (0, 1):
                b = bb + p
                row0 = base + b * _RB
                pltpu.make_async_copy(
                    kmers_hbm.at[pl.ds(row0, _RB)], kbs[p], sis[p]).wait()

                @pl.when(b >= 2)
                def _wait_out():
                    pltpu.make_async_copy(
                        hs[p], out_hbm.at[pl.ds(row0 - 2 * _RB, _RB)],
                        sos[p]).wait()

                for r2 in range(_RB):
                    rsplat = jnp.full((_LANES,), r2, jnp.int32)

                    @plsc.parallel_loop(0, _NBINS, _LANES, unroll=8)
                    def _zero(i):
                        hs[p][r2, pl.ds(i, _LANES)] = zeros16

                    @plsc.parallel_loop(0, _TAIL_START, _LANES, unroll=8)
                    def _chunk(i):
                        idx = kbs[p][r2, pl.ds(i, _LANES)]
                        cnt, last = plsc.scan_count(idx)
                        plsc.addupdate_scatter(
                            hs[p], [rsplat, idx],
                            cnt.astype(jnp.float32), mask=last)

                    idx = kbs[p][r2, pl.ds(_TAIL_START, _LANES)]
                    cnt, last = plsc.scan_count(idx, mask=tail_valid)
                    plsc.addupdate_scatter(
                        hs[p], [rsplat, idx],
                        cnt.astype(jnp.float32), mask=last)

                pltpu.async_copy(hs[p], out_hbm.at[pl.ds(row0, _RB)], sos[p])

                @pl.when(b + 2 < _NBATCH)
                def _next_in():
                    pltpu.async_copy(
                        kmers_hbm.at[pl.ds(row0 + 2 * _RB, _RB)],
                        kbs[p], sis[p])

        pltpu.make_async_copy(
            h0, out_hbm.at[pl.ds(base + (_NBATCH - 2) * _RB, _RB)], so0).wait()
        pltpu.make_async_copy(
            h1, out_hbm.at[pl.ds(base + (_NBATCH - 1) * _RB, _RB)], so1).wait()

    return k(kmers)


def kernel(sequence):
    xt = jnp.transpose(sequence, (0, 2, 1))
    kmers = _kmer_tc(xt)
    return _hist_sc(kmers)


# speedup vs baseline: 1.0331x; 1.0331x over previous
import dataclasses
import functools

import jax
import jax.numpy as jnp
from jax import lax
from jax.experimental import pallas as pl
from jax.experimental.pallas import tpu as pltpu
from jax.experimental.pallas import tpu_sc as plsc

_A = 4
_K = 6
_B = 1024
_L = 4096
_NBINS = _A ** _K
_OUT_LEN = _L - _K + 1

_R1 = 16

_NW = 32
_ROWS_PER_WORKER = _B // _NW

_LANES = 16
_TAIL_START = (_OUT_LEN // _LANES) * _LANES
_TAIL_N = _OUT_LEN - _TAIL_START

_RB = 4
_NBATCH = _ROWS_PER_WORKER // _RB


def _shup(x, d, axis):
    return pltpu.roll(x, x.shape[axis] - d, axis)


def _kmer_tc_body(x_ref, o_ref):
    x0 = x_ref[:, 0, :]
    x1 = x_ref[:, 1, :]
    x2 = x_ref[:, 2, :]
    x3 = x_ref[:, 3, :]
    m01 = jnp.maximum(x0, x1)
    b01 = m01 > x0
    m23 = jnp.maximum(x2, x3)
    h = m23 > m01
    i01 = jnp.where(b01, 1, 0)
    i23 = jnp.where(m23 > x2, 3, 2)
    c = jnp.where(h, i23, i01)
    y1 = c * 4 + _shup(c, 1, 1)
    y2 = y1 * 16 + _shup(y1, 2, 1)
    km = y2 * 16 + _shup(y1, 4, 1)
    o_ref[...] = km


_NCHUNK = 2
_CB = _B // _NCHUNK


def _kmer_tc(xt, c0, nrows):
    return pl.pallas_call(
        _kmer_tc_body,
        grid=(nrows // _R1,),
        in_specs=[pl.BlockSpec((_R1, _A, _L),
                               lambda i, c=c0: (i + c // _R1, 0, 0))],
        out_specs=pl.BlockSpec((_R1, _L), lambda i: (i, 0)),
        out_shape=jax.ShapeDtypeStruct((nrows, _L), jnp.int32),
        compiler_params=pltpu.CompilerParams(
            dimension_semantics=("parallel",)),
    )(xt)


def _hist_sc(kmers):
    mesh = plsc.VectorSubcoreMesh(core_axis_name="c", subcore_axis_name="s")
    cp = pltpu.CompilerParams()
    if "needs_layout_passes" in pltpu.CompilerParams.__dataclass_fields__:
        cp = dataclasses.replace(cp, needs_layout_passes=False)

    @functools.partial(
        pl.kernel,
        compiler_params=cp,
        out_type=jax.ShapeDtypeStruct((_B, _NBINS), jnp.float32),
        mesh=mesh,
        scratch_types=[
            pltpu.VMEM((_RB, _L), jnp.int32),
            pltpu.VMEM((_RB, _L), jnp.int32),
            pltpu.VMEM((_RB, _NBINS), jnp.float32),
            pltpu.VMEM((_RB, _NBINS), jnp.float32),
            pltpu.SemaphoreType.DMA,
            pltpu.SemaphoreType.DMA,
            pltpu.SemaphoreType.DMA,
            pltpu.SemaphoreType.DMA,
        ],
    )
    def k(kmers_hbm, out_hbm, kb0, kb1, h0, h1, si0, si1, so0, so1):
        wid = lax.axis_index("s") * 2 + lax.axis_index("c")
        base = wid * _ROWS_PER_WORKER
        kbs = (kb0, kb1)
        hs = (h0, h1)
        sis = (si0, si1)
        sos = (so0, so1)
        zeros16 = jnp.zeros((_LANES,), jnp.float32)
        tail_valid = lax.iota(jnp.int32, _LANES) < _TAIL_N

        pltpu.async_copy(kmers_hbm.at[pl.ds(base, _RB)], kb0, si0)
        pltpu.async_copy(kmers_hbm.at[pl.ds(base + _RB, _RB)], kb1, si1)

        @pl.loop(0, _NBATCH, step=2)
        def _batch(bb):
            for p in (0, 1):
                b = bb + p
                row0 = base + b * _RB
                pltpu.make_async_copy(
                    kmers_hbm.at[pl.ds(row0, _RB)], kbs[p], sis[p]).wait()

                @pl.when(b >= 2)
                def _wait_out():
                    pltpu.make_async_copy(
                        hs[p], out_hbm.at[pl.ds(row0 - 2 * _RB, _RB)],
                        sos[p]).wait()

                for r2 in range(_RB):
                    rsplat = jnp.full((_LANES,), r2, jnp.int32)

                    @plsc.parallel_loop(0, _NBINS, _LANES, unroll=8)
                    def _zero(i):
                        hs[p][r2, pl.ds(i, _LANES)] = zeros16

                    @plsc.parallel_loop(0, _TAIL_START, _LANES, unroll=8)
                    def _chunk(i):
                        idx = kbs[p][r2, pl.ds(i, _LANES)]
                        cnt, last = plsc.scan_count(idx)
                        plsc.addupdate_scatter(
                            hs[p], [rsplat, idx],
                            cnt.astype(jnp.float32), mask=last)

                    idx = kbs[p][r2, pl.ds(_TAIL_START, _LANES)]
                    cnt, last = plsc.scan_count(idx, mask=tail_valid)
                    plsc.addupdate_scatter(
                        hs[p], [rsplat, idx],
                        cnt.astype(jnp.float32), mask=last)

                pltpu.async_copy(hs[p], out_hbm.at[pl.ds(row0, _RB)], sos[p])

                @pl.when(b + 2 < _NBATCH)
                def _next_in():
                    pltpu.async_copy(
                        kmers_hbm.at[pl.ds(row0 + 2 * _RB, _RB)],
                        kbs[p], sis[p])

        pltpu.make_async_copy(
            h0, out_hbm.at[pl.ds(base + (_NBATCH - 2) * _RB, _RB)], so0).wait()
        pltpu.make_async_copy(
            h1, out_hbm.at[pl.ds(base + (_NBATCH - 1) * _RB, _RB)], so1).wait()

    return k(kmers)


def kernel(sequence):
    xt = jnp.transpose(sequence, (0, 2, 1))
    kmers = _kmer_tc(xt)
    return _hist_sc(kmers)


# speedup vs baseline: 1.0579x; 1.0239x over previous
import dataclasses
import functools

import jax
import jax.numpy as jnp
from jax import lax
from jax.experimental import pallas as pl
from jax.experimental.pallas import tpu as pltpu
from jax.experimental.pallas import tpu_sc as plsc

_A = 4
_K = 6
_B = 1024
_L = 4096
_NBINS = _A ** _K
_OUT_LEN = _L - _K + 1

_R1 = 16

_NW = 32
_ROWS_PER_WORKER = _B // _NW

_LANES = 16
_TAIL_START = (_OUT_LEN // _LANES) * _LANES
_TAIL_N = _OUT_LEN - _TAIL_START

_RB = 4
_NBATCH = _ROWS_PER_WORKER // _RB


def _shup(x, d, axis):
    return pltpu.roll(x, x.shape[axis] - d, axis)


def _kmer_tc_body(x_ref, o_ref):
    x0 = x_ref[:, 0, :]
    x1 = x_ref[:, 1, :]
    x2 = x_ref[:, 2, :]
    x3 = x_ref[:, 3, :]
    m01 = jnp.maximum(x0, x1)
    b01 = m01 > x0
    m23 = jnp.maximum(x2, x3)
    h = m23 > m01
    i01 = jnp.where(b01, 1, 0)
    i23 = jnp.where(m23 > x2, 3, 2)
    c = jnp.where(h, i23, i01)
    y1 = c * 4 + _shup(c, 1, 1)
    y2 = y1 * 16 + _shup(y1, 2, 1)
    km = y2 * 16 + _shup(y1, 4, 1)
    o_ref[...] = km


_NCHUNK = 2
_CB = _B // _NCHUNK


def _kmer_tc(xt, c0, nrows):
    return pl.pallas_call(
        _kmer_tc_body,
        grid=(nrows // _R1,),
        in_specs=[pl.BlockSpec((_R1, _A, _L),
                               lambda i, c=c0: (i + c // _R1, 0, 0))],
        out_specs=pl.BlockSpec((_R1, _L), lambda i: (i, 0)),
        out_shape=jax.ShapeDtypeStruct((nrows, _L), jnp.int32),
        compiler_params=pltpu.CompilerParams(
            dimension_semantics=("parallel",)),
    )(xt)


def _hist_sc(kmers):
    mesh = plsc.VectorSubcoreMesh(core_axis_name="c", subcore_axis_name="s")
    cp = pltpu.CompilerParams()
    if "needs_layout_passes" in pltpu.CompilerParams.__dataclass_fields__:
        cp = dataclasses.replace(cp, needs_layout_passes=False)

    @functools.partial(
        pl.kernel,
        compiler_params=cp,
        out_type=jax.ShapeDtypeStruct((_B, _NBINS), jnp.float32),
        mesh=mesh,
        scratch_types=[
            pltpu.VMEM((_RB, _L), jnp.int32),
            pltpu.VMEM((_RB, _L), jnp.int32),
            pltpu.VMEM((_RB, _NBINS), jnp.float32),
            pltpu.VMEM((_RB, _NBINS), jnp.float32),
            pltpu.SemaphoreType.DMA,
            pltpu.SemaphoreType.DMA,
            pltpu.SemaphoreType.DMA,
            pltpu.SemaphoreType.DMA,
        ],
    )
    def k(kmers_hbm, out_hbm, kb0, kb1, h0, h1, si0, si1, so0, so1):
        wid = lax.axis_index("s") * 2 + lax.axis_index("c")
        base = wid * _ROWS_PER_WORKER
        kbs = (kb0, kb1)
        hs = (h0, h1)
        sis = (si0, si1)
        sos = (so0, so1)
        zeros16 = jnp.zeros((_LANES,), jnp.float32)
        ones16 = jnp.ones((_LANES,), jnp.float32)
        tail_valid = lax.iota(jnp.int32, _LANES) < _TAIL_N

        pltpu.async_copy(kmers_hbm.at[pl.ds(base, _RB)], kb0, si0)
        pltpu.async_copy(kmers_hbm.at[pl.ds(base + _RB, _RB)], kb1, si1)

        @pl.loop(0, _NBATCH, step=2)
        def _batch(bb):
            for p in (0, 1):
                b = bb + p
                row0 = base + b * _RB
                pltpu.make_async_copy(
                    kmers_hbm.at[pl.ds(row0, _RB)], kbs[p], sis[p]).wait()

                @pl.when(b >= 2)
                def _wait_out():
                    pltpu.make_async_copy(
                        hs[p], out_hbm.at[pl.ds(row0 - 2 * _RB, _RB)],
                        sos[p]).wait()

                for r2 in range(_RB):
                    rsplat = jnp.full((_LANES,), r2, jnp.int32)

                    @plsc.parallel_loop(0, _NBINS, _LANES, unroll=8)
                    def _zero(i):
                        hs[p][r2, pl.ds(i, _LANES)] = zeros16

                    @plsc.parallel_loop(0, _TAIL_START, _LANES, unroll=8)
                    def _chunk(i):
                        idx = kbs[p][r2, pl.ds(i, _LANES)]
                        plsc.addupdate_scatter(
                            hs[p], [rsplat, idx], ones16)

                    idx = kbs[p][r2, pl.ds(_TAIL_START, _LANES)]
                    plsc.addupdate_scatter(
                        hs[p], [rsplat, idx], ones16, mask=tail_valid)

                pltpu.async_copy(hs[p], out_hbm.at[pl.ds(row0, _RB)], sos[p])

                @pl.when(b + 2 < _NBATCH)
                def _next_in():
                    pltpu.async_copy(
                        kmers_hbm.at[pl.ds(row0 + 2 * _RB, _RB)],
                        kbs[p], sis[p])

        pltpu.make_async_copy(
            h0, out_hbm.at[pl.ds(base + (_NBATCH - 2) * _RB, _RB)], so0).wait()
        pltpu.make_async_copy(
            h1, out_hbm.at[pl.ds(base + (_NBATCH - 1) * _RB, _RB)], so1).wait()

    return k(kmers)


def kernel(sequence):
    xt = jnp.transpose(sequence, (0, 2, 1))
    kmers = _kmer_tc(xt)
    return _hist_sc(kmers)
